# baseline (speedup 1.0000x reference)
import jax
import jax.numpy as jnp
from jax.experimental import pallas as pl


_BS = 256


def _add_kernel(x_ref, t_ref, o_ref):
    o_ref[...] = x_ref[...] + t_ref[...][None, :, :]


def kernel(inputs, pos_table):
    batch, seq, dm = inputs.shape
    nblk = seq // _BS
    return pl.pallas_call(
        _add_kernel,
        grid=(nblk,),
        in_specs=[
            pl.BlockSpec((batch, _BS, dm), lambda i: (0, i, 0)),
            pl.BlockSpec((_BS, dm), lambda i: (i, 0)),
        ],
        out_specs=pl.BlockSpec((batch, _BS, dm), lambda i: (0, i, 0)),
        out_shape=jax.ShapeDtypeStruct(inputs.shape, inputs.dtype),
    )(inputs, pos_table)
